# baseline (speedup 1.0000x reference)
import functools

import jax
import jax.numpy as jnp
from jax import lax
from jax.experimental import pallas as pl
from jax.experimental.pallas import tpu as pltpu
from jax.experimental.pallas import tpu_sc as plsc

NUM_EMBED = 1000000
EMBED_DIM = 64
BATCH = 16384
HIST = 50
B_TOTAL = BATCH * HIST

_info = plsc.get_sparse_core_info()
NC, NS = _info.num_cores, _info.num_subcores
NW = NC * NS
B_PER_W = B_TOTAL // NW
CHUNK = 128
NCHUNK = B_PER_W // CHUNK
NBUF = 8
DEPTH = 4


def _make_kernel():
    mesh = plsc.VectorSubcoreMesh(core_axis_name="c", subcore_axis_name="s")

    @functools.partial(
        pl.kernel,
        mesh=mesh,
        out_type=jax.ShapeDtypeStruct((B_TOTAL, EMBED_DIM), jnp.float32),
        compiler_params=pltpu.CompilerParams(use_tc_tiling_on_sc=False),
        scratch_types=[
            pltpu.VMEM((NCHUNK, CHUNK), jnp.int32),
            [pltpu.VMEM((CHUNK, EMBED_DIM), jnp.float32) for _ in range(NBUF)],
            [pltpu.SemaphoreType.DMA for _ in range(NBUF)],
            [pltpu.SemaphoreType.DMA for _ in range(NBUF)],
        ],
    )
    def k(table_hbm, idx_hbm, out_hbm, idx_v, rows, gsems, psems):
        wid = lax.axis_index("s") * NC + lax.axis_index("c")
        base = wid * B_PER_W
        pltpu.sync_copy(idx_hbm.at[wid], idx_v)

        def fire_gather(j, s):
            pltpu.async_copy(table_hbm.at[idx_v.at[j]], rows[s], gsems[s])

        def drain_gather(s):
            pltpu.make_async_copy(
                table_hbm.at[pl.ds(0, CHUNK)], rows[s], gsems[s]
            ).wait()

        def fire_put(j, s):
            pltpu.async_copy(
                rows[s], out_hbm.at[pl.ds(base + j * CHUNK, CHUNK)], psems[s]
            )

        def drain_put(s):
            pltpu.make_async_copy(
                table_hbm.at[pl.ds(0, CHUNK)], rows[s], psems[s]
            ).wait()

        for j in range(DEPTH):
            fire_gather(j, j)

        def body(t, carry):
            for phase in range(NBUF):
                j = t * NBUF + phase
                s = phase
                sn = (phase + DEPTH) % NBUF
                jn = j + DEPTH

                @pl.when(jn < NCHUNK)
                def _():
                    @pl.when(jn >= NBUF)
                    def _():
                        drain_put(sn)

                    fire_gather(jn, sn)

                drain_gather(s)
                fire_put(j, s)

            return carry

        lax.fori_loop(0, NCHUNK // NBUF, body, 0)
        for s in range(NBUF):
            drain_put(s)

    return k


_sc_gather = _make_kernel()


def kernel(inputs, vec_matrix):
    idx = inputs.reshape(NW, NCHUNK, CHUNK).astype(jnp.int32)
    out = _sc_gather(vec_matrix, idx)
    return out.reshape(BATCH, HIST, EMBED_DIM)
